# baseline (speedup 1.0000x reference)
import jax
import jax.numpy as jnp
from jax import lax
from jax.experimental import pallas as pl
from jax.experimental.pallas import tpu as pltpu
from jax.experimental.pallas import tpu_sc as plsc

V = 100000
H = 128
SEG = 2
AGE = 120
GEN = 3
ETH = 10
INS = 5
P = 512
B = 1024
S = 200
EPS = 1e-12

NC = 2
NS = 16
L = 16
NW = NC * NS
N = B * S
TPW = N // NW
CH = 64
NCHUNK = TPW // CH
KG = H // L
NCAT = SEG * GEN * ETH * INS


def _rsqrt(x):
    i = lax.bitcast_convert_type(x, jnp.int32)
    y = lax.bitcast_convert_type(jnp.int32(0x5F3759DF) - (i >> 1), jnp.float32)
    for _ in range(2):
        y = y * (1.5 - 0.5 * x * y * y)
    return y


def _sc_body(word_ids, age_ids, seg_ids, gen_ids, eth_ids, ins_ids, posi_ids,
             word_w, ac_w, posi_w, gamma, beta,
             out,
             widx, aidx, sidx, gidx, eidx, iidx, pidx, acidx,
             wrows, acrows, posi_v, gb, statsf, mrbuf,
             sem_idx, sem_gat, sem_out):
    wid = lax.axis_index("s") * NC + lax.axis_index("c")
    base = wid * TPW

    pltpu.sync_copy(gamma, gb.at[0])
    pltpu.sync_copy(beta, gb.at[1])
    pltpu.sync_copy(posi_w, posi_v)
    gvec = [gb[0, pl.ds(k * L, L)] for k in range(KG)]
    bvec = [gb[1, pl.ds(k * L, L)] for k in range(KG)]

    def stage_idx(g, b):
        off = base + g * CH
        pltpu.async_copy(word_ids.at[pl.ds(off, CH)], widx.at[b], sem_idx)
        pltpu.async_copy(posi_ids.at[pl.ds(off, CH)], pidx.at[b], sem_idx)
        for src, dst in ((age_ids, aidx), (seg_ids, sidx), (gen_ids, gidx),
                         (eth_ids, eidx), (ins_ids, iidx)):
            pltpu.async_copy(src.at[pl.ds(off, CH)], dst, sem_idx)

    def wait_idx(b):
        pltpu.make_async_copy(word_ids.at[pl.ds(0, CH)], widx.at[b],
                              sem_idx).wait()
        pltpu.make_async_copy(posi_ids.at[pl.ds(0, CH)], pidx.at[b],
                              sem_idx).wait()
        for dst in (aidx, sidx, gidx, eidx, iidx):
            pltpu.make_async_copy(word_ids.at[pl.ds(0, CH)], dst,
                                  sem_idx).wait()

    def fuse_cats(b):
        for g in range(CH // L):
            sl = pl.ds(g * L, L)
            cc = ((sidx[sl] * GEN + gidx[sl]) * ETH + eidx[sl]) * INS \
                + iidx[sl]
            acidx[b, sl] = aidx[sl] * NCAT + cc

    def issue_gathers(b):
        pltpu.async_copy(word_w.at[widx.at[b]], wrows.at[b], sem_gat)
        pltpu.async_copy(ac_w.at[acidx.at[b]], acrows.at[b], sem_gat)

    def wait_gathers(b):
        pltpu.make_async_copy(word_w.at[pl.ds(0, CH)], wrows.at[b], sem_gat).wait()
        pltpu.make_async_copy(ac_w.at[pl.ds(0, CH)], acrows.at[b], sem_gat).wait()

    def wait_scatter(b):
        pltpu.make_async_copy(wrows.at[b], out.at[pl.ds(0, CH)], sem_out).wait()

    def compute_chunk(b):
        iota_ch = lax.iota(jnp.int32, L) * CH
        iota_l = lax.iota(jnp.int32, L)

        @plsc.parallel_loop(0, CH, unroll=2)
        def token_sum(t):
            tg = jnp.bitwise_and(t, jnp.int32(~(L - 1)))
            lane = jnp.bitwise_and(t, jnp.int32(L - 1))
            pvec = pidx[b, pl.ds(tg, L)]
            pid = jnp.take_along_axis(
                pvec, jnp.full((L,), lane, dtype=jnp.int32), axis=0)
            paddr = pid * H + iota_l
            acc = None
            acc2 = None
            for k in range(KG):
                sl = pl.ds(k * L, L)
                pk = plsc.load_gather(posi_v, [paddr + (k * L)])
                x = (wrows[b, t, sl] + acrows[b, t, sl]) + pk
                wrows[b, t, sl] = x
                acc = x if acc is None else acc + x
                acc2 = x * x if acc2 is None else acc2 + x * x
            idx_a = iota_ch + jnp.full((L,), t, dtype=jnp.int32)
            plsc.store_scatter(statsf, [idx_a], acc)
            plsc.store_scatter(statsf, [idx_a + (L * CH)], acc2)

        def group_body(g, gc):
            s = None
            s2 = None
            for lane in range(L):
                va = statsf[pl.ds(lane * CH + g * L, L)]
                s = va if s is None else s + va
            for lane in range(L):
                vb = statsf[pl.ds((L + lane) * CH + g * L, L)]
                s2 = vb if s2 is None else s2 + vb
            mean_vec = s * (1.0 / H)
            msq_vec = s2 * (1.0 / H)
            var_vec = jnp.maximum(msq_vec - mean_vec * mean_vec, 0.0) + EPS
            rstd_vec = _rsqrt(var_vec)
            mrbuf[0, pl.ds(g * L, L)] = mean_vec
            mrbuf[1, pl.ds(g * L, L)] = rstd_vec
            return gc

        lax.fori_loop(0, CH // L, group_body, 0)

        @plsc.parallel_loop(0, CH, unroll=2)
        def token_norm(t):
            tg = jnp.bitwise_and(t, jnp.int32(~(L - 1)))
            lane = jnp.bitwise_and(t, jnp.int32(L - 1))
            lane_splat = jnp.full((L,), lane, dtype=jnp.int32)
            mean_v = jnp.take_along_axis(mrbuf[0, pl.ds(tg, L)], lane_splat,
                                         axis=0)
            rstd_v = jnp.take_along_axis(mrbuf[1, pl.ds(tg, L)], lane_splat,
                                         axis=0)
            for k in range(KG):
                sl = pl.ds(k * L, L)
                xh = (wrows[b, t, sl] - mean_v) * rstd_v
                wrows[b, t, sl] = xh * gvec[k] + bvec[k]

    stage_idx(0, 0)
    wait_idx(0)
    fuse_cats(0)
    issue_gathers(0)
    stage_idx(1, 1)
    wait_idx(1)
    fuse_cats(1)
    issue_gathers(1)
    stage_idx(2, 2)

    def step(g, bsel):
        nb2 = (bsel + 2) % 3

        @pl.when(g + 2 < NCHUNK)
        def _prefetch():
            wait_idx(nb2)
            fuse_cats(nb2)

            @pl.when(g >= 1)
            def _():
                wait_scatter(nb2)

            issue_gathers(nb2)

        wait_gathers(bsel)
        compute_chunk(bsel)
        pltpu.async_copy(wrows.at[bsel], out.at[pl.ds(base + g * CH, CH)],
                         sem_out)

        @pl.when(g + 3 < NCHUNK)
        def _():
            stage_idx(g + 3, bsel)

    def triple_body(g3, carry):
        for bsel in range(3):
            step(g3 * 3 + bsel, bsel)
        return carry

    lax.fori_loop(0, NCHUNK // 3, triple_body, 0, unroll=False)
    step(jnp.int32(NCHUNK - 1), 0)
    wait_scatter(1)
    wait_scatter(2)
    wait_scatter(0)


@jax.jit
def _run(word_ids, age_ids, seg_ids, gen_ids, eth_ids, ins_ids, posi_ids,
         word_w, ac_w, posi_w, gamma, beta):
    mesh = plsc.VectorSubcoreMesh(core_axis_name="c", subcore_axis_name="s")
    f = pl.kernel(
        _sc_body,
        out_type=jax.ShapeDtypeStruct((N, H), jnp.float32),
        mesh=mesh,
        compiler_params=pltpu.CompilerParams(needs_layout_passes=False),
        scratch_types=[
            pltpu.VMEM((3, CH), jnp.int32),
            pltpu.VMEM((CH,), jnp.int32),
            pltpu.VMEM((CH,), jnp.int32),
            pltpu.VMEM((CH,), jnp.int32),
            pltpu.VMEM((CH,), jnp.int32),
            pltpu.VMEM((CH,), jnp.int32),
            pltpu.VMEM((3, CH), jnp.int32),
            pltpu.VMEM((3, CH), jnp.int32),
            pltpu.VMEM((3, CH, H), jnp.float32),
            pltpu.VMEM((3, CH, H), jnp.float32),
            pltpu.VMEM((P * H,), jnp.float32),
            pltpu.VMEM((2, H), jnp.float32),
            pltpu.VMEM((2 * L * CH,), jnp.float32),
            pltpu.VMEM((2, CH), jnp.float32),
            pltpu.SemaphoreType.DMA,
            pltpu.SemaphoreType.DMA,
            pltpu.SemaphoreType.DMA,
        ],
    )
    return f(word_ids, age_ids, seg_ids, gen_ids, eth_ids, ins_ids, posi_ids,
             word_w, ac_w, posi_w, gamma, beta)


def kernel(word_ids, age_ids, gender_ids, ethni_ids, ins_ids, seg_ids, posi_ids,
           word_w, seg_w, age_w, gender_w, ethni_w, ins_w, posi_w, ln_gamma, ln_beta):
    cat_w = (seg_w[:, None, None, None, :] + gender_w[None, :, None, None, :]
             + ethni_w[None, None, :, None, :] + ins_w[None, None, None, :, :]
             ).reshape(NCAT, H)
    ac_w = (age_w[:, None, :] + cat_w[None, :, :]).reshape(AGE * NCAT, H)
    flat = lambda x: x.reshape(N).astype(jnp.int32)
    out = _run(flat(word_ids), flat(age_ids), flat(seg_ids), flat(gender_ids),
               flat(ethni_ids), flat(ins_ids), flat(posi_ids),
               word_w, ac_w, posi_w.reshape(P * H), ln_gamma, ln_beta)
    return out.reshape(B, S, H)


# speedup vs baseline: 1.2365x; 1.2365x over previous
import jax
import jax.numpy as jnp
from jax import lax
from jax.experimental import pallas as pl
from jax.experimental.pallas import tpu as pltpu
from jax.experimental.pallas import tpu_sc as plsc

V = 100000
H = 128
SEG = 2
AGE = 120
GEN = 3
ETH = 10
INS = 5
P = 512
B = 1024
S = 200
EPS = 1e-12

NC = 2
NS = 16
L = 16
NW = NC * NS
N = B * S
TPW = N // NW
CH = 80
NCHUNK = TPW // CH
KG = H // L
NCAT = SEG * GEN * ETH * INS


def _rsqrt(x):
    i = lax.bitcast_convert_type(x, jnp.int32)
    y = lax.bitcast_convert_type(jnp.int32(0x5F3759DF) - (i >> 1), jnp.float32)
    for _ in range(2):
        y = y * (1.5 - 0.5 * x * y * y)
    return y


def _sc_body(word_ids, age_ids, seg_ids, gen_ids, eth_ids, ins_ids, posi_ids,
             word_w, ac_w, posi_w, gamma, beta,
             out,
             widx, aidx, sidx, gidx, eidx, iidx, pidx, acidx,
             wrows, acrows, posi_v, gb, statsf, mrbuf,
             sem_idx, sem_gat, sem_out):
    wid = lax.axis_index("s") * NC + lax.axis_index("c")
    base = wid * TPW

    pltpu.sync_copy(gamma, gb.at[0])
    pltpu.sync_copy(beta, gb.at[1])
    pltpu.sync_copy(posi_w, posi_v)
    gvec = [gb[0, pl.ds(k * L, L)] for k in range(KG)]
    bvec = [gb[1, pl.ds(k * L, L)] for k in range(KG)]

    idx_pairs = ((word_ids, widx), (age_ids, aidx), (seg_ids, sidx),
                 (gen_ids, gidx), (eth_ids, eidx), (ins_ids, iidx),
                 (posi_ids, pidx))

    def stage_idx(g, b):
        off = base + g * CH
        for src, dst in idx_pairs:
            pltpu.async_copy(src.at[pl.ds(off, CH)], dst.at[b], sem_idx)

    def wait_idx(b):
        for src, dst in idx_pairs:
            pltpu.make_async_copy(src.at[pl.ds(0, CH)], dst.at[b], sem_idx).wait()

    def fuse_cats(b):
        for g in range(CH // L):
            sl = pl.ds(g * L, L)
            cc = ((sidx[b, sl] * GEN + gidx[b, sl]) * ETH + eidx[b, sl]) * INS \
                + iidx[b, sl]
            acidx[b, sl] = aidx[b, sl] * NCAT + cc

    def issue_gathers(b):
        pltpu.async_copy(word_w.at[widx.at[b]], wrows.at[b], sem_gat)
        pltpu.async_copy(ac_w.at[acidx.at[b]], acrows.at[b], sem_gat)

    def wait_gathers(b):
        pltpu.make_async_copy(word_w.at[pl.ds(0, CH)], wrows.at[b], sem_gat).wait()
        pltpu.make_async_copy(ac_w.at[pl.ds(0, CH)], acrows.at[b], sem_gat).wait()

    def wait_scatter(b):
        pltpu.make_async_copy(wrows.at[b], out.at[pl.ds(0, CH)], sem_out).wait()

    def compute_chunk(b):
        iota_ch = lax.iota(jnp.int32, L) * CH
        iota_l = lax.iota(jnp.int32, L)

        @plsc.parallel_loop(0, CH, unroll=2)
        def token_sum(t):
            tg = jnp.bitwise_and(t, jnp.int32(~(L - 1)))
            lane = jnp.bitwise_and(t, jnp.int32(L - 1))
            pvec = pidx[b, pl.ds(tg, L)]
            pid = jnp.take_along_axis(
                pvec, jnp.full((L,), lane, dtype=jnp.int32), axis=0)
            paddr = pid * H + iota_l
            acc = None
            acc2 = None
            for k in range(KG):
                sl = pl.ds(k * L, L)
                pk = plsc.load_gather(posi_v, [paddr + (k * L)])
                x = (wrows[b, t, sl] + acrows[b, t, sl]) + pk
                wrows[b, t, sl] = x
                acc = x if acc is None else acc + x
                acc2 = x * x if acc2 is None else acc2 + x * x
            idx_a = iota_ch + jnp.full((L,), t, dtype=jnp.int32)
            plsc.store_scatter(statsf, [idx_a], acc)
            plsc.store_scatter(statsf, [idx_a + (L * CH)], acc2)

        def group_body(g, gc):
            s = None
            s2 = None
            for lane in range(L):
                va = statsf[pl.ds(lane * CH + g * L, L)]
                s = va if s is None else s + va
            for lane in range(L):
                vb = statsf[pl.ds((L + lane) * CH + g * L, L)]
                s2 = vb if s2 is None else s2 + vb
            mean_vec = s * (1.0 / H)
            msq_vec = s2 * (1.0 / H)
            var_vec = jnp.maximum(msq_vec - mean_vec * mean_vec, 0.0) + EPS
            rstd_vec = _rsqrt(var_vec)
            mrbuf[0, pl.ds(g * L, L)] = mean_vec
            mrbuf[1, pl.ds(g * L, L)] = rstd_vec
            return gc

        lax.fori_loop(0, CH // L, group_body, 0)

        @plsc.parallel_loop(0, CH, unroll=2)
        def token_norm(t):
            tg = jnp.bitwise_and(t, jnp.int32(~(L - 1)))
            lane = jnp.bitwise_and(t, jnp.int32(L - 1))
            lane_splat = jnp.full((L,), lane, dtype=jnp.int32)
            mean_v = jnp.take_along_axis(mrbuf[0, pl.ds(tg, L)], lane_splat,
                                         axis=0)
            rstd_v = jnp.take_along_axis(mrbuf[1, pl.ds(tg, L)], lane_splat,
                                         axis=0)
            for k in range(KG):
                sl = pl.ds(k * L, L)
                xh = (wrows[b, t, sl] - mean_v) * rstd_v
                wrows[b, t, sl] = xh * gvec[k] + bvec[k]

    stage_idx(0, 0)
    wait_idx(0)
    fuse_cats(0)
    issue_gathers(0)
    stage_idx(1, 1)

    def pair_body(g2, carry):
        for bsel in range(2):
            g = g2 * 2 + bsel
            nb = 1 - bsel

            @pl.when(g + 1 < NCHUNK)
            def _prefetch():
                wait_idx(nb)
                fuse_cats(nb)

                @pl.when(g >= 1)
                def _():
                    wait_scatter(nb)

                issue_gathers(nb)

            wait_gathers(bsel)
            compute_chunk(bsel)
            pltpu.async_copy(wrows.at[bsel], out.at[pl.ds(base + g * CH, CH)],
                             sem_out)

            @pl.when(g + 2 < NCHUNK)
            def _():
                stage_idx(g + 2, bsel)
        return carry

    lax.fori_loop(0, NCHUNK // 2, pair_body, 0, unroll=False)
    wait_scatter(1)


@jax.jit
def _run(word_ids, age_ids, seg_ids, gen_ids, eth_ids, ins_ids, posi_ids,
         word_w, ac_w, posi_w, gamma, beta):
    mesh = plsc.VectorSubcoreMesh(core_axis_name="c", subcore_axis_name="s")
    f = pl.kernel(
        _sc_body,
        out_type=jax.ShapeDtypeStruct((N, H), jnp.float32),
        mesh=mesh,
        compiler_params=pltpu.CompilerParams(needs_layout_passes=False),
        scratch_types=[
            pltpu.VMEM((2, CH), jnp.int32),
            pltpu.VMEM((2, CH), jnp.int32),
            pltpu.VMEM((2, CH), jnp.int32),
            pltpu.VMEM((2, CH), jnp.int32),
            pltpu.VMEM((2, CH), jnp.int32),
            pltpu.VMEM((2, CH), jnp.int32),
            pltpu.VMEM((2, CH), jnp.int32),
            pltpu.VMEM((2, CH), jnp.int32),
            pltpu.VMEM((2, CH, H), jnp.float32),
            pltpu.VMEM((2, CH, H), jnp.float32),
            pltpu.VMEM((P * H,), jnp.float32),
            pltpu.VMEM((2, H), jnp.float32),
            pltpu.VMEM((2 * L * CH,), jnp.float32),
            pltpu.VMEM((2, CH), jnp.float32),
            pltpu.SemaphoreType.DMA,
            pltpu.SemaphoreType.DMA,
            pltpu.SemaphoreType.DMA,
        ],
    )
    return f(word_ids, age_ids, seg_ids, gen_ids, eth_ids, ins_ids, posi_ids,
             word_w, ac_w, posi_w, gamma, beta)


def kernel(word_ids, age_ids, gender_ids, ethni_ids, ins_ids, seg_ids, posi_ids,
           word_w, seg_w, age_w, gender_w, ethni_w, ins_w, posi_w, ln_gamma, ln_beta):
    cat_w = (seg_w[:, None, None, None, :] + gender_w[None, :, None, None, :]
             + ethni_w[None, None, :, None, :] + ins_w[None, None, None, :, :]
             ).reshape(NCAT, H)
    ac_w = (age_w[:, None, :] + cat_w[None, :, :]).reshape(AGE * NCAT, H)
    flat = lambda x: x.reshape(N).astype(jnp.int32)
    out = _run(flat(word_ids), flat(age_ids), flat(seg_ids), flat(gender_ids),
               flat(ethni_ids), flat(ins_ids), flat(posi_ids),
               word_w, ac_w, posi_w.reshape(P * H), ln_gamma, ln_beta)
    return out.reshape(B, S, H)
